# Initial kernel scaffold; baseline (speedup 1.0000x reference)
import jax
import jax.numpy as jnp
from jax.experimental import pallas as pl


def kernel(x, neg_labels, emb_predictor, emb_cf_perm, emb_cf_primary, emb_cf_secondary, emb_reorder, gw_predictor__cf_perm, gb_predictor__cf_perm, gw_predictor__cf_primary, gb_predictor__cf_primary, gw_predictor__cf_secondary, gb_predictor__cf_secondary, gw_reorder__cf_perm, gb_reorder__cf_perm, gw_reorder__cf_primary, gb_reorder__cf_primary, gw_reorder__cf_secondary, gb_reorder__cf_secondary):
    raise NotImplementedError("write your pallas kernel here")


# speedup vs baseline: 1.3033x; 1.3033x over previous
import functools

import jax
import jax.numpy as jnp
from jax import lax
from jax.experimental import pallas as pl
from jax.experimental.pallas import tpu as pltpu
from jax.experimental.pallas import tpu_sc as plsc

_B = 1024
_NNEG = 50
_DIM = 64
_IN_DIM = 128
_NPAIR = 6
_NHEAD = 5

_NP = 56
_NC = 2
_NS = 16
_NW = _NC * _NS
_BPW = _B // _NW


def _gates_tc(x, gw, gb):
    def body(x_ref, w_ref, b_ref, o_ref):
        o_ref[...] = jnp.tanh(
            jnp.dot(x_ref[...], w_ref[...], preferred_element_type=jnp.float32,
                    precision=lax.Precision.HIGHEST)
            + b_ref[...]
        )
    return pl.pallas_call(
        body,
        out_shape=jax.ShapeDtypeStruct((_B, _NPAIR * _DIM), jnp.float32),
    )(x, gw, gb)


def _make_sc_kernel():
    mesh = plsc.VectorSubcoreMesh(core_axis_name="c", subcore_axis_name="s")

    @functools.partial(
        pl.kernel,
        out_type=jax.ShapeDtypeStruct((_B, _NP), jnp.float32),
        mesh=mesh,
        scratch_types=[
            pltpu.VMEM((_NHEAD, _NP), jnp.int32),
            pltpu.VMEM((_NPAIR, _DIM), jnp.float32),
            pltpu.VMEM((_NHEAD, _NP, _DIM), jnp.float32),
            pltpu.VMEM((64, 16), jnp.float32),
            pltpu.VMEM((64,), jnp.float32),
            pltpu.SemaphoreType.DMA,
        ],
        compiler_params=pltpu.CompilerParams(
            needs_layout_passes=False, use_tc_tiling_on_sc=False),
    )
    def sc(labels_hbm, gates_hbm, t0, t1, t2, t3, t4, out_hbm,
           lab_v, gate_v, rows_v, accs_v, score_v, sem):
        wid = lax.axis_index("s") * _NC + lax.axis_index("c")
        zero16 = jnp.zeros((16,), jnp.float32)
        for r in range(_NP, 64):
            accs_v[r] = zero16

        tables = (t0, t1, t2, t3, t4)
        lanes = lax.iota(jnp.int32, 16)

        def batch_body(i, carry):
            b = wid * _BPW + i
            pltpu.sync_copy(labels_hbm.at[b], lab_v)
            pltpu.sync_copy(gates_hbm.at[b], gate_v)
            cps = [
                pltpu.async_copy(tables[h].at[lab_v.at[h]], rows_v.at[h], sem)
                for h in range(_NHEAD)
            ]
            for cp in cps:
                cp.wait()

            for c in range(4):
                sl = pl.ds(c * 16, 16)
                g0 = gate_v[0, sl]
                g1 = gate_v[1, sl]
                g2 = gate_v[2, sl]
                g3 = gate_v[3, sl]
                g4 = gate_v[4, sl]
                g5 = gate_v[5, sl]

                def neg_body(n, _, c=c, sl=sl, g0=g0, g1=g1, g2=g2,
                             g3=g3, g4=g4, g5=g5):
                    pred = rows_v[0, n, sl]
                    perm = rows_v[1, n, sl]
                    prim = rows_v[2, n, sl]
                    sec = rows_v[3, n, sl]
                    reo = rows_v[4, n, sl]
                    a1 = pred * g0 + reo * g3
                    a2 = pred * g1 + reo * g4
                    a3 = pred * g2 + reo * g5
                    contrib = a1 * perm + a2 * prim + a3 * sec
                    if c == 0:
                        accs_v[n] = contrib
                    else:
                        plsc.addupdate(accs_v.at[n], contrib)
                    return 0

                lax.fori_loop(0, _NP, neg_body, 0)

            for grp in range(4):
                tot = zero16
                for j in range(16):
                    s = jnp.sum(accs_v[grp * 16 + j])
                    tot = jnp.where(lanes == j, s, tot)
                score_v[pl.ds(grp * 16, 16)] = tot

            pltpu.sync_copy(score_v.at[pl.ds(0, _NP)], out_hbm.at[b])
            return carry

        lax.fori_loop(0, _BPW, batch_body, 0)

    return sc


_sc_kernel = _make_sc_kernel()


def kernel(x, neg_labels, emb_predictor, emb_cf_perm, emb_cf_primary,
           emb_cf_secondary, emb_reorder,
           gw_predictor__cf_perm, gb_predictor__cf_perm,
           gw_predictor__cf_primary, gb_predictor__cf_primary,
           gw_predictor__cf_secondary, gb_predictor__cf_secondary,
           gw_reorder__cf_perm, gb_reorder__cf_perm,
           gw_reorder__cf_primary, gb_reorder__cf_primary,
           gw_reorder__cf_secondary, gb_reorder__cf_secondary):
    gw = jnp.concatenate(
        [gw_predictor__cf_perm, gw_predictor__cf_primary,
         gw_predictor__cf_secondary, gw_reorder__cf_perm,
         gw_reorder__cf_primary, gw_reorder__cf_secondary], axis=1)
    gb = jnp.concatenate(
        [gb_predictor__cf_perm, gb_predictor__cf_primary,
         gb_predictor__cf_secondary, gb_reorder__cf_perm,
         gb_reorder__cf_primary, gb_reorder__cf_secondary], axis=0)
    gates = _gates_tc(x, gw, gb.reshape(1, _NPAIR * _DIM))
    gates = gates.reshape(_B, _NPAIR, _DIM)

    lab = jnp.transpose(neg_labels[:, :, :_NHEAD], (0, 2, 1))
    lab = jnp.pad(lab, ((0, 0), (0, 0), (0, _NP - _NNEG)))

    score = _sc_kernel(lab, gates, emb_predictor, emb_cf_perm,
                       emb_cf_primary, emb_cf_secondary, emb_reorder)
    return score[:, :_NNEG]
